# baseline (speedup 1.0000x reference)
import functools

import jax
import jax.numpy as jnp
from jax import lax
from jax.experimental import pallas as pl
from jax.experimental.pallas import tpu as pltpu
from jax.experimental.pallas import tpu_sc as plsc

N_MACHINE = 10000
N_JOB = 50000
E = 320000
D_FEAT = 128
HID = 64
HEADS = 3
HC = HID * HEADS
EDGE_DIM = 5
NUM_LAYERS = 2

FW = 256
AW = 208
NTILES = 32
PER_TILE = E // NTILES
EB = 80
S3_SB = 2000

_CP = pltpu.CompilerParams(needs_layout_passes=False)


def _mesh():
    return plsc.VectorSubcoreMesh(core_axis_name="c", subcore_axis_name="s")


def _mm_body(x_ref, w_ref, b_ref, o_ref, *, act, act_in):
    x = x_ref[...]
    if act_in == "tanh":
        x = jnp.tanh(x)
    y = jnp.dot(x, w_ref[...], preferred_element_type=jnp.float32)
    y = y + b_ref[...]
    if act == "tanh":
        y = jnp.tanh(y)
    o_ref[...] = y


def _mm(x, w, b, act=None, act_in=None, bm=1024):
    m, k = x.shape
    _, n = w.shape
    grid = (pl.cdiv(m, bm),)
    return pl.pallas_call(
        functools.partial(_mm_body, act=act, act_in=act_in),
        grid=grid,
        in_specs=[
            pl.BlockSpec((bm, k), lambda i: (i, 0)),
            pl.BlockSpec((k, n), lambda i: (0, 0)),
            pl.BlockSpec((1, n), lambda i: (0, 0)),
        ],
        out_specs=pl.BlockSpec((bm, n), lambda i: (i, 0)),
        out_shape=jax.ShapeDtypeStruct((m, n), jnp.float32),
    )(x, w, b.reshape(1, n))


def _mm_pad(x, w, b, **kw):
    _, n = w.shape
    wp = jnp.pad(w, ((0, 0), (0, FW - n)))
    bp = jnp.pad(b, (0, FW - n))
    return _mm(x, wp, bp, **kw)


def _edge_body(x_ref, w_ref, s_ref, d_ref, o_ref):
    y = jnp.dot(x_ref[...], w_ref[...], preferred_element_type=jnp.float32)
    z = jnp.zeros((y.shape[0], FW - HC - 2), jnp.float32)
    o_ref[...] = jnp.concatenate([y, s_ref[...], d_ref[...], z], axis=1)


def _edge_rows(ea, we, src_f, dst_f, bm=1024):
    grid = (pl.cdiv(E, bm),)
    return pl.pallas_call(
        _edge_body,
        grid=grid,
        in_specs=[
            pl.BlockSpec((bm, EDGE_DIM), lambda i: (i, 0)),
            pl.BlockSpec((EDGE_DIM, HC), lambda i: (0, 0)),
            pl.BlockSpec((bm, 1), lambda i: (i, 0)),
            pl.BlockSpec((bm, 1), lambda i: (i, 0)),
        ],
        out_specs=pl.BlockSpec((bm, FW), lambda i: (i, 0)),
        out_shape=jax.ShapeDtypeStruct((E, FW), jnp.float32),
    )(ea, we, src_f, dst_f)


def _norm_body(a_ref, fb_ref, fd_ref, b_ref, o_ref, *, bm):
    i = pl.program_id(0)
    a = a_ref[...]
    fd = fd_ref[...]
    fb = fb_ref[...]
    rows = jax.lax.broadcasted_iota(jnp.int32, (bm, 1), 0) + i * bm
    for t in range(NTILES):
        fd_t = fd[0, t * 16]
        if t == 0:
            shared = jnp.bool_(False)
        else:
            shared = fd[0, (t - 1) * 16 + 1] == fd_t
        hit = rows == fd_t
        a = jnp.where(hit & (~shared), fb[t:t + 1, :], a)
        a = a + (hit & shared).astype(jnp.float32) * fb[t:t + 1, :]
    x = a[:, :HC]
    parts = []
    for h in range(HEADS):
        dn = a[:, HC + h:HC + h + 1]
        parts.append(x[:, h * HID:(h + 1) * HID] / (dn + 1e-16))
    o_ref[...] = jnp.concatenate(parts, axis=1) + b_ref[...]


def _norm(acc, fb, fd, bias, n_out, bm=512):
    grid = (pl.cdiv(n_out, bm),)
    return pl.pallas_call(
        functools.partial(_norm_body, bm=bm),
        grid=grid,
        in_specs=[
            pl.BlockSpec((bm, AW), lambda i: (i, 0)),
            pl.BlockSpec((NTILES, AW), lambda i: (0, 0)),
            pl.BlockSpec((1, NTILES * 16), lambda i: (0, 0)),
            pl.BlockSpec((1, HC), lambda i: (0, 0)),
        ],
        out_specs=pl.BlockSpec((bm, HC), lambda i: (i, 0)),
        out_shape=jax.ShapeDtypeStruct((n_out, HC), jnp.float32),
    )(acc, fb, fd.reshape(1, NTILES * 16), bias.reshape(1, HC))


def _s2(xl, xr, eesd, perm, att, n_dst):
    NBLK = PER_TILE // EB
    NS = jnp.float32(0.2)

    @functools.partial(
        pl.kernel,
        mesh=_mesh(),
        compiler_params=_CP,
        out_type=[
            jax.ShapeDtypeStruct((n_dst, AW), jnp.float32),
            jax.ShapeDtypeStruct((NTILES, AW), jnp.float32),
            jax.ShapeDtypeStruct((NTILES * 16,), jnp.int32),
        ],
        scratch_types=[
            pltpu.VMEM((EB, FW), jnp.float32),
            pltpu.VMEM((EB, FW), jnp.float32),
            pltpu.VMEM((2 * EB, FW), jnp.float32),
            pltpu.VMEM((2 * EB,), jnp.int32),
            pltpu.VMEM((EB + 16,), jnp.int32),
            pltpu.VMEM((EB + 16,), jnp.int32),
            pltpu.VMEM((HC,), jnp.float32),
            pltpu.VMEM((1, AW), jnp.float32),
            pltpu.VMEM((16,), jnp.int32),
            pltpu.SemaphoreType.DMA,
            pltpu.SemaphoreType.DMA,
        ],
    )
    def k(xl_h, xr_h, ee_h, perm_h, att_h, acc_h, fb_h, fd_h,
          xlv, xrv, eev, eidv, srcv, dstv, attv, ar, fdv, sem, sem2):
        cid = lax.axis_index("c")
        sid = lax.axis_index("s")
        wid = cid * 16 + sid
        base = wid * PER_TILE
        zero = jnp.zeros((16,), jnp.float32)
        lane = lax.iota(jnp.int32, 16)

        for q in range(AW // 16):
            ar[0, pl.ds(q * 16, 16)] = zero
        pltpu.sync_copy(att_h, attv)

        def prefetch(b, poff):
            g0 = base + b * EB
            pltpu.sync_copy(perm_h.at[pl.ds(g0, EB)],
                            eidv.at[pl.ds(poff, EB)])
            pltpu.async_copy(ee_h.at[eidv.at[pl.ds(poff, EB)]],
                             eev.at[pl.ds(poff, EB), :], sem2)

        def finish_block(poff):
            pltpu.make_async_copy(ee_h.at[pl.ds(0, EB), :],
                                  eev.at[pl.ds(poff, EB), :], sem2).wait()
            def xb(j, _):
                rows16 = poff + j * 16 + lane
                sf = plsc.load_gather(
                    eev, [rows16, jnp.broadcast_to(jnp.int32(HC), (16,))])
                df = plsc.load_gather(
                    eev, [rows16, jnp.broadcast_to(jnp.int32(HC + 1), (16,))])
                srcv[pl.ds(j * 16, 16)] = sf.astype(jnp.int32)
                dstv[pl.ds(j * 16, 16)] = df.astype(jnp.int32)
                return _

            lax.fori_loop(0, EB // 16, xb, None)
            c1 = pltpu.async_copy(xl_h.at[srcv.at[pl.ds(0, EB)]], xlv, sem)
            c2 = pltpu.async_copy(xr_h.at[dstv.at[pl.ds(0, EB)]], xrv, sem)
            return c1, c2

        prefetch(0, 0)
        c1, c2 = finish_block(0)
        prefetch(1, EB)
        c1.wait()
        c2.wait()
        prev0 = dstv[pl.ds(0, 16)][0]
        fdv[pl.ds(0, 16)] = jnp.where(lane == 0, prev0, 0)

        def blk_body(b, carry):
            @pl.when(b > 0)
            def _():
                poff = (b & 1) * EB
                c1, c2 = finish_block(poff)

                @pl.when(b + 1 < NBLK)
                def _():
                    prefetch(b + 1, EB - poff)

                c1.wait()
                c2.wait()

            poff = (b & 1) * EB

            def edge_body(e2, c):
                for sub_e in range(2):
                    c = _one_edge(e2 * 2 + sub_e, c)
                return c

            def _one_edge(e, c):
                prev, fdone = c
                d = dstv[pl.ds(e, 16)][0]
                flush = d != prev

                @pl.when(flush & (fdone == 1))
                def _():
                    pltpu.sync_copy(ar, acc_h.at[pl.ds(prev, 1), :])

                @pl.when(flush & (fdone == 0))
                def _():
                    pltpu.sync_copy(ar, fb_h.at[pl.ds(wid, 1), :])

                @pl.when(flush)
                def _():
                    for q in range(AW // 16):
                        ar[0, pl.ds(q * 16, 16)] = zero

                exs = []
                for h in range(HEADS):
                    acc = jnp.zeros((16,), jnp.float32)
                    for q in range(HID // 16):
                        o = h * HID + q * 16
                        mq = (xlv[e, pl.ds(o, 16)]
                              + xrv[e, pl.ds(o, 16)]
                              + eev[poff + e, pl.ds(o, 16)])
                        g = jnp.where(mq >= 0, mq, NS * mq)
                        acc = acc + g * attv[pl.ds(o, 16)]
                    sh = jnp.sum(acc)
                    exv = jnp.exp(jnp.broadcast_to(sh, (16,)))
                    for q in range(HID // 16):
                        o = h * HID + q * 16
                        plsc.addupdate(ar.at[0, pl.ds(o, 16)],
                                       exv * xlv[e, pl.ds(o, 16)])
                    exs.append(exv)
                exrow = jnp.where(
                    lane == 0, exs[0],
                    jnp.where(lane == 1, exs[1],
                              jnp.where(lane == 2, exs[2],
                                        jnp.float32(0.0))))
                plsc.addupdate(ar.at[0, pl.ds(HC, 16)], exrow)
                return (d, jnp.where(flush, jnp.int32(1), fdone))

            return lax.fori_loop(0, EB // 2, edge_body, carry)

        prev, fdone = lax.fori_loop(0, NBLK, blk_body,
                                    (prev0, jnp.int32(0)))

        @pl.when(fdone == 1)
        def _():
            pltpu.sync_copy(ar, acc_h.at[pl.ds(prev, 1), :])

        @pl.when(fdone == 0)
        def _():
            pltpu.sync_copy(ar, fb_h.at[pl.ds(wid, 1), :])

        fdv[pl.ds(0, 16)] = jnp.where(lane == 1, prev, fdv[pl.ds(0, 16)])
        pltpu.sync_copy(fdv, fd_h.at[pl.ds(wid * 16, 16)])

    return k(xl, xr, eesd, perm, att)


def _s3(esrc, edst, se, sm, sj):
    NSUB = PER_TILE // S3_SB
    NGRP = S3_SB // 16

    @functools.partial(
        pl.kernel,
        mesh=_mesh(),
        compiler_params=_CP,
        out_type=jax.ShapeDtypeStruct((E,), jnp.float32),
        scratch_types=[
            pltpu.VMEM((N_MACHINE,), jnp.float32),
            pltpu.VMEM((N_JOB,), jnp.float32),
            pltpu.VMEM((S3_SB,), jnp.int32),
            pltpu.VMEM((S3_SB,), jnp.int32),
            pltpu.VMEM((S3_SB,), jnp.float32),
            pltpu.VMEM((S3_SB,), jnp.float32),
        ],
    )
    def k(esrc_h, edst_h, se_h, sm_h, sj_h, out_h,
          smv, sjv, srcv, dstv, sev, outv):
        wid = lax.axis_index("c") * 16 + lax.axis_index("s")
        base = wid * PER_TILE
        pltpu.sync_copy(sm_h, smv)
        pltpu.sync_copy(sj_h, sjv)
        for sub in range(NSUB):
            off = base + sub * S3_SB
            pltpu.sync_copy(esrc_h.at[pl.ds(off, S3_SB)], srcv)
            pltpu.sync_copy(edst_h.at[pl.ds(off, S3_SB)], dstv)
            pltpu.sync_copy(se_h.at[pl.ds(off, S3_SB)], sev)

            def gbody(g, _):
                s16 = srcv[pl.ds(g * 16, 16)]
                d16 = dstv[pl.ds(g * 16, 16)]
                v = (plsc.load_gather(smv, [s16])
                     + plsc.load_gather(sjv, [d16])
                     + sev[pl.ds(g * 16, 16)])
                outv[pl.ds(g * 16, 16)] = v
                return _

            lax.fori_loop(0, NGRP, gbody, None)
            pltpu.sync_copy(outv, out_h.at[pl.ds(off, S3_SB)])

    return k(esrc, edst, se, sm, sj)


def kernel(x_machine, x_job, edge_src, edge_dst, edge_attr_exec, edge_attr_rev,
           params):
    p = params
    z192 = jnp.zeros((HC,), jnp.float32)
    xm = _mm(x_machine, p["W1_m"], p["b1_m"], act="tanh")
    xj = _mm(x_job, p["W1_j"], p["b1_j"], act="tanh")

    perm_e = jnp.argsort(edge_dst).astype(jnp.int32)
    perm_r = jnp.argsort(edge_src).astype(jnp.int32)
    src_f = edge_src.astype(jnp.float32).reshape(E, 1)
    dst_f = edge_dst.astype(jnp.float32).reshape(E, 1)

    for l in range(NUM_LAYERS):
        xl_e = _mm_pad(xm, p[f"Wl{l}e"], z192)
        xr_e = _mm_pad(xj, p[f"Wr{l}e"], z192)
        ee_e = _edge_rows(edge_attr_exec, p[f"We{l}e"], src_f, dst_f)
        xl_r = _mm_pad(xj, p[f"Wl{l}r"], z192)
        xr_r = _mm_pad(xm, p[f"Wr{l}r"], z192)
        ee_r = _edge_rows(edge_attr_rev, p[f"We{l}r"], dst_f, src_f)
        accj, fbj, fdj = _s2(xl_e, xr_e, ee_e, perm_e,
                             p[f"att{l}e"].reshape(HC), N_JOB)
        accm, fbm, fdm = _s2(xl_r, xr_r, ee_r, perm_r,
                             p[f"att{l}r"].reshape(HC), N_MACHINE)
        xj = _norm(accj, fbj, fdj, p[f"b{l}e"], N_JOB)
        xm = _norm(accm, fbm, fdm, p[f"b{l}r"], N_MACHINE)

    W3, b3 = p["W3"], p["b3"]
    z8 = jnp.zeros((8,), jnp.float32)
    wm = jnp.pad(W3[:HC], ((0, 0), (0, 7)))
    wj = jnp.pad(W3[HC + EDGE_DIM:], ((0, 0), (0, 7)))
    we = jnp.pad(W3[HC:HC + EDGE_DIM], ((0, 0), (0, 7)))
    b3p = jnp.concatenate([b3, jnp.zeros((7,), jnp.float32)])
    sm = _mm(xm, wm, z8, act_in="tanh")[:, 0]
    sj = _mm(xj, wj, z8, act_in="tanh")[:, 0]
    se = _mm(edge_attr_exec, we, b3p)[:, 0]
    score = _s3(edge_src, edge_dst, se, sm, sj)
    return score.reshape(E, 1)


# speedup vs baseline: 1.0020x; 1.0020x over previous
import functools

import jax
import jax.numpy as jnp
from jax import lax
from jax.experimental import pallas as pl
from jax.experimental.pallas import tpu as pltpu
from jax.experimental.pallas import tpu_sc as plsc

N_MACHINE = 10000
N_JOB = 50000
E = 320000
D_FEAT = 128
HID = 64
HEADS = 3
HC = HID * HEADS
EDGE_DIM = 5
NUM_LAYERS = 2

FW = 256
AW = 208
NTILES = 32
PER_TILE = E // NTILES
EB = 80
S3_SB = 2000

_CP = pltpu.CompilerParams(needs_layout_passes=False)


def _mesh():
    return plsc.VectorSubcoreMesh(core_axis_name="c", subcore_axis_name="s")


def _mm_body(x_ref, w_ref, b_ref, o_ref, *, act, act_in):
    x = x_ref[...]
    if act_in == "tanh":
        x = jnp.tanh(x)
    y = jnp.dot(x, w_ref[...], preferred_element_type=jnp.float32)
    y = y + b_ref[...]
    if act == "tanh":
        y = jnp.tanh(y)
    o_ref[...] = y


def _mm(x, w, b, act=None, act_in=None, bm=1024):
    m, k = x.shape
    _, n = w.shape
    grid = (pl.cdiv(m, bm),)
    return pl.pallas_call(
        functools.partial(_mm_body, act=act, act_in=act_in),
        grid=grid,
        in_specs=[
            pl.BlockSpec((bm, k), lambda i: (i, 0)),
            pl.BlockSpec((k, n), lambda i: (0, 0)),
            pl.BlockSpec((1, n), lambda i: (0, 0)),
        ],
        out_specs=pl.BlockSpec((bm, n), lambda i: (i, 0)),
        out_shape=jax.ShapeDtypeStruct((m, n), jnp.float32),
    )(x, w, b.reshape(1, n))


def _mm_pad(x, w, b, **kw):
    _, n = w.shape
    wp = jnp.pad(w, ((0, 0), (0, FW - n)))
    bp = jnp.pad(b, (0, FW - n))
    return _mm(x, wp, bp, **kw)


def _edge_body(x_ref, w_ref, s_ref, d_ref, o_ref):
    y = jnp.dot(x_ref[...], w_ref[...], preferred_element_type=jnp.float32)
    z = jnp.zeros((y.shape[0], FW - HC - 2), jnp.float32)
    o_ref[...] = jnp.concatenate([y, s_ref[...], d_ref[...], z], axis=1)


def _edge_rows(ea, we, src_f, dst_f, bm=1024):
    grid = (pl.cdiv(E, bm),)
    return pl.pallas_call(
        _edge_body,
        grid=grid,
        in_specs=[
            pl.BlockSpec((bm, EDGE_DIM), lambda i: (i, 0)),
            pl.BlockSpec((EDGE_DIM, HC), lambda i: (0, 0)),
            pl.BlockSpec((bm, 1), lambda i: (i, 0)),
            pl.BlockSpec((bm, 1), lambda i: (i, 0)),
        ],
        out_specs=pl.BlockSpec((bm, FW), lambda i: (i, 0)),
        out_shape=jax.ShapeDtypeStruct((E, FW), jnp.float32),
    )(ea, we, src_f, dst_f)


def _norm_body(a_ref, fb_ref, fd_ref, b_ref, o_ref, *, bm):
    i = pl.program_id(0)
    a = a_ref[...]
    fd = fd_ref[...]
    fb = fb_ref[...]
    rows = jax.lax.broadcasted_iota(jnp.int32, (bm, 1), 0) + i * bm
    for t in range(NTILES):
        fd_t = fd[0, t * 16]
        if t == 0:
            shared = jnp.bool_(False)
        else:
            shared = fd[0, (t - 1) * 16 + 1] == fd_t
        hit = rows == fd_t
        a = jnp.where(hit & (~shared), fb[t:t + 1, :], a)
        a = a + (hit & shared).astype(jnp.float32) * fb[t:t + 1, :]
    x = a[:, :HC]
    parts = []
    for h in range(HEADS):
        dn = a[:, HC + h:HC + h + 1]
        parts.append(x[:, h * HID:(h + 1) * HID] / (dn + 1e-16))
    o_ref[...] = jnp.concatenate(parts, axis=1) + b_ref[...]


def _norm(acc, fb, fd, bias, n_out, bm=512):
    grid = (pl.cdiv(n_out, bm),)
    return pl.pallas_call(
        functools.partial(_norm_body, bm=bm),
        grid=grid,
        in_specs=[
            pl.BlockSpec((bm, AW), lambda i: (i, 0)),
            pl.BlockSpec((NTILES, AW), lambda i: (0, 0)),
            pl.BlockSpec((1, NTILES * 16), lambda i: (0, 0)),
            pl.BlockSpec((1, HC), lambda i: (0, 0)),
        ],
        out_specs=pl.BlockSpec((bm, HC), lambda i: (i, 0)),
        out_shape=jax.ShapeDtypeStruct((n_out, HC), jnp.float32),
    )(acc, fb, fd.reshape(1, NTILES * 16), bias.reshape(1, HC))


def _s2(xl, xr, eesd, perm, att, n_dst):
    NBLK = PER_TILE // EB
    NS = jnp.float32(0.2)

    @functools.partial(
        pl.kernel,
        mesh=_mesh(),
        compiler_params=_CP,
        out_type=[
            jax.ShapeDtypeStruct((n_dst, AW), jnp.float32),
            jax.ShapeDtypeStruct((NTILES, AW), jnp.float32),
            jax.ShapeDtypeStruct((NTILES * 16,), jnp.int32),
        ],
        scratch_types=[
            pltpu.VMEM((EB, FW), jnp.float32),
            pltpu.VMEM((EB, FW), jnp.float32),
            pltpu.VMEM((2 * EB, FW), jnp.float32),
            pltpu.VMEM((2 * EB,), jnp.int32),
            pltpu.VMEM((EB + 16,), jnp.int32),
            pltpu.VMEM((EB + 16,), jnp.int32),
            pltpu.VMEM((HC,), jnp.float32),
            pltpu.VMEM((1, AW), jnp.float32),
            pltpu.VMEM((16,), jnp.int32),
            pltpu.SemaphoreType.DMA,
            pltpu.SemaphoreType.DMA,
        ],
    )
    def k(xl_h, xr_h, ee_h, perm_h, att_h, acc_h, fb_h, fd_h,
          xlv, xrv, eev, eidv, srcv, dstv, attv, ar, fdv, sem, sem2):
        cid = lax.axis_index("c")
        sid = lax.axis_index("s")
        wid = cid * 16 + sid
        base = wid * PER_TILE
        zero = jnp.zeros((16,), jnp.float32)
        lane = lax.iota(jnp.int32, 16)

        for q in range(AW // 16):
            ar[0, pl.ds(q * 16, 16)] = zero
        pltpu.sync_copy(att_h, attv)

        def prefetch(b, poff):
            g0 = base + b * EB
            pltpu.sync_copy(perm_h.at[pl.ds(g0, EB)],
                            eidv.at[pl.ds(poff, EB)])
            pltpu.async_copy(ee_h.at[eidv.at[pl.ds(poff, EB)]],
                             eev.at[pl.ds(poff, EB), :], sem2)

        def finish_block(poff):
            pltpu.make_async_copy(ee_h.at[pl.ds(0, EB), :],
                                  eev.at[pl.ds(poff, EB), :], sem2).wait()
            def xb(j, _):
                rows16 = poff + j * 16 + lane
                sf = plsc.load_gather(
                    eev, [rows16, jnp.broadcast_to(jnp.int32(HC), (16,))])
                df = plsc.load_gather(
                    eev, [rows16, jnp.broadcast_to(jnp.int32(HC + 1), (16,))])
                srcv[pl.ds(j * 16, 16)] = sf.astype(jnp.int32)
                dstv[pl.ds(j * 16, 16)] = df.astype(jnp.int32)
                return _

            lax.fori_loop(0, EB // 16, xb, None)
            c1 = pltpu.async_copy(xl_h.at[srcv.at[pl.ds(0, EB)]], xlv, sem)
            c2 = pltpu.async_copy(xr_h.at[dstv.at[pl.ds(0, EB)]], xrv, sem)
            return c1, c2

        prefetch(0, 0)
        c1, c2 = finish_block(0)
        prefetch(1, EB)
        c1.wait()
        c2.wait()
        prev0 = dstv[pl.ds(0, 16)][0]
        fdv[pl.ds(0, 16)] = jnp.where(lane == 0, prev0, 0)

        def blk_body(b, carry):
            @pl.when(b > 0)
            def _():
                poff = (b & 1) * EB
                c1, c2 = finish_block(poff)

                @pl.when(b + 1 < NBLK)
                def _():
                    prefetch(b + 1, EB - poff)

                c1.wait()
                c2.wait()

            poff = (b & 1) * EB

            def edge_body(e, c):
                prev, fdone = c
                d = dstv[pl.ds(e, 16)][0]
                flush = d != prev

                @pl.when(flush & (fdone == 1))
                def _():
                    pltpu.sync_copy(ar, acc_h.at[pl.ds(prev, 1), :])

                @pl.when(flush & (fdone == 0))
                def _():
                    pltpu.sync_copy(ar, fb_h.at[pl.ds(wid, 1), :])

                @pl.when(flush)
                def _():
                    for q in range(AW // 16):
                        ar[0, pl.ds(q * 16, 16)] = zero

                exs = []
                for h in range(HEADS):
                    acc = jnp.zeros((16,), jnp.float32)
                    for q in range(HID // 16):
                        o = h * HID + q * 16
                        mq = (xlv[e, pl.ds(o, 16)]
                              + xrv[e, pl.ds(o, 16)]
                              + eev[poff + e, pl.ds(o, 16)])
                        g = jnp.where(mq >= 0, mq, NS * mq)
                        acc = acc + g * attv[pl.ds(o, 16)]
                    sh = jnp.sum(acc)
                    exv = jnp.exp(jnp.broadcast_to(sh, (16,)))
                    for q in range(HID // 16):
                        o = h * HID + q * 16
                        plsc.addupdate(ar.at[0, pl.ds(o, 16)],
                                       exv * xlv[e, pl.ds(o, 16)])
                    exs.append(exv)
                exrow = jnp.where(
                    lane == 0, exs[0],
                    jnp.where(lane == 1, exs[1],
                              jnp.where(lane == 2, exs[2],
                                        jnp.float32(0.0))))
                plsc.addupdate(ar.at[0, pl.ds(HC, 16)], exrow)
                return (d, jnp.where(flush, jnp.int32(1), fdone))

            return lax.fori_loop(0, EB, edge_body, carry)

        prev, fdone = lax.fori_loop(0, NBLK, blk_body,
                                    (prev0, jnp.int32(0)))

        @pl.when(fdone == 1)
        def _():
            pltpu.sync_copy(ar, acc_h.at[pl.ds(prev, 1), :])

        @pl.when(fdone == 0)
        def _():
            pltpu.sync_copy(ar, fb_h.at[pl.ds(wid, 1), :])

        fdv[pl.ds(0, 16)] = jnp.where(lane == 1, prev, fdv[pl.ds(0, 16)])
        pltpu.sync_copy(fdv, fd_h.at[pl.ds(wid * 16, 16)])

    return k(xl, xr, eesd, perm, att)


def _s3(esrc, edst, se, sm, sj):
    NSUB = PER_TILE // S3_SB
    NGRP = S3_SB // 16

    @functools.partial(
        pl.kernel,
        mesh=_mesh(),
        compiler_params=_CP,
        out_type=jax.ShapeDtypeStruct((E,), jnp.float32),
        scratch_types=[
            pltpu.VMEM((N_MACHINE,), jnp.float32),
            pltpu.VMEM((N_JOB,), jnp.float32),
            pltpu.VMEM((S3_SB,), jnp.int32),
            pltpu.VMEM((S3_SB,), jnp.int32),
            pltpu.VMEM((S3_SB,), jnp.float32),
            pltpu.VMEM((S3_SB,), jnp.float32),
        ],
    )
    def k(esrc_h, edst_h, se_h, sm_h, sj_h, out_h,
          smv, sjv, srcv, dstv, sev, outv):
        wid = lax.axis_index("c") * 16 + lax.axis_index("s")
        base = wid * PER_TILE
        pltpu.sync_copy(sm_h, smv)
        pltpu.sync_copy(sj_h, sjv)
        for sub in range(NSUB):
            off = base + sub * S3_SB
            pltpu.sync_copy(esrc_h.at[pl.ds(off, S3_SB)], srcv)
            pltpu.sync_copy(edst_h.at[pl.ds(off, S3_SB)], dstv)
            pltpu.sync_copy(se_h.at[pl.ds(off, S3_SB)], sev)

            def gbody(g, _):
                s16 = srcv[pl.ds(g * 16, 16)]
                d16 = dstv[pl.ds(g * 16, 16)]
                v = (plsc.load_gather(smv, [s16])
                     + plsc.load_gather(sjv, [d16])
                     + sev[pl.ds(g * 16, 16)])
                outv[pl.ds(g * 16, 16)] = v
                return _

            lax.fori_loop(0, NGRP, gbody, None)
            pltpu.sync_copy(outv, out_h.at[pl.ds(off, S3_SB)])

    return k(esrc, edst, se, sm, sj)


def kernel(x_machine, x_job, edge_src, edge_dst, edge_attr_exec, edge_attr_rev,
           params):
    p = params
    z192 = jnp.zeros((HC,), jnp.float32)
    xm = _mm(x_machine, p["W1_m"], p["b1_m"], act="tanh")
    xj = _mm(x_job, p["W1_j"], p["b1_j"], act="tanh")

    perm_e = jnp.argsort(edge_dst).astype(jnp.int32)
    perm_r = jnp.argsort(edge_src).astype(jnp.int32)
    src_f = edge_src.astype(jnp.float32).reshape(E, 1)
    dst_f = edge_dst.astype(jnp.float32).reshape(E, 1)

    for l in range(NUM_LAYERS):
        xl_e = _mm_pad(xm, p[f"Wl{l}e"], z192)
        xr_e = _mm_pad(xj, p[f"Wr{l}e"], z192)
        ee_e = _edge_rows(edge_attr_exec, p[f"We{l}e"], src_f, dst_f)
        xl_r = _mm_pad(xj, p[f"Wl{l}r"], z192)
        xr_r = _mm_pad(xm, p[f"Wr{l}r"], z192)
        ee_r = _edge_rows(edge_attr_rev, p[f"We{l}r"], dst_f, src_f)
        accj, fbj, fdj = _s2(xl_e, xr_e, ee_e, perm_e,
                             p[f"att{l}e"].reshape(HC), N_JOB)
        accm, fbm, fdm = _s2(xl_r, xr_r, ee_r, perm_r,
                             p[f"att{l}r"].reshape(HC), N_MACHINE)
        xj = _norm(accj, fbj, fdj, p[f"b{l}e"], N_JOB)
        xm = _norm(accm, fbm, fdm, p[f"b{l}r"], N_MACHINE)

    W3, b3 = p["W3"], p["b3"]
    z8 = jnp.zeros((8,), jnp.float32)
    wm = jnp.pad(W3[:HC], ((0, 0), (0, 7)))
    wj = jnp.pad(W3[HC + EDGE_DIM:], ((0, 0), (0, 7)))
    we = jnp.pad(W3[HC:HC + EDGE_DIM], ((0, 0), (0, 7)))
    b3p = jnp.concatenate([b3, jnp.zeros((7,), jnp.float32)])
    sm = _mm(xm, wm, z8, act_in="tanh")[:, 0]
    sj = _mm(xj, wj, z8, act_in="tanh")[:, 0]
    se = _mm(edge_attr_exec, we, b3p)[:, 0]
    score = _s3(edge_src, edge_dst, se, sm, sj)
    return score.reshape(E, 1)
